# Initial kernel scaffold; baseline (speedup 1.0000x reference)
import jax
import jax.numpy as jnp
from jax.experimental import pallas as pl


def kernel(features, indices, labels):
    raise NotImplementedError("write your pallas kernel here")


# speedup vs baseline: 5.2319x; 5.2319x over previous
import functools

import jax
import jax.numpy as jnp
from jax import lax
from jax.experimental import pallas as pl
from jax.experimental.pallas import tpu as pltpu
from jax.experimental.pallas import tpu_sc as plsc


def _prep_body(lab_row_ref, lab_col_ref, idx_row_ref, idx_col_ref,
               keep_ref, minv_ref):
    lab_row = lab_row_ref[...]
    lab_col = lab_col_ref[...]
    idx_row = idx_row_ref[...]
    idx_col = idx_col_ref[...]
    b = lab_row.shape[1]
    eq_lab = lab_col == lab_row
    eq_idx = idx_col == idx_row
    ia = lax.broadcasted_iota(jnp.int32, (b, b), 0)
    ij = lax.broadcasted_iota(jnp.int32, (b, b), 1)
    dup = eq_lab & eq_idx & (ia < ij)
    keep = jnp.where(jnp.any(dup, axis=0, keepdims=True), 0.0, 1.0)
    m = jnp.sum(eq_lab.astype(jnp.float32) * keep, axis=1, keepdims=True)
    keep_ref[...] = keep
    minv_ref[...] = 1.0 / m


def _prep(labels, indices):
    b = labels.shape[0]
    keep, minv = pl.pallas_call(
        _prep_body,
        out_shape=[jax.ShapeDtypeStruct((1, b), jnp.float32),
                   jax.ShapeDtypeStruct((b, 1), jnp.float32)],
    )(labels.reshape(1, b), labels.reshape(b, 1),
      indices.reshape(1, b), indices.reshape(b, 1))
    return keep.reshape(b), minv.reshape(b)


def _make_sc_gather(b, v):
    info = plsc.get_sparse_core_info()
    nc, ns, lanes = info.num_cores, info.num_subcores, info.num_lanes
    nw = nc * ns
    rpw = b // nw
    nch = b // lanes
    nstream = b // 128
    mesh = plsc.VectorSubcoreMesh(core_axis_name="c", subcore_axis_name="s")

    @functools.partial(
        pl.kernel, mesh=mesh,
        out_type=jax.ShapeDtypeStruct((nw, lanes), jnp.float32),
        scratch_types=[
            pltpu.VMEM((b,), jnp.int32),
            pltpu.VMEM((b + 16,), jnp.int32),
            pltpu.VMEM((b,), jnp.float32),
            pltpu.VMEM((b + 16,), jnp.float32),
            pltpu.VMEM((nstream, 128), jnp.int32),
            pltpu.VMEM((nstream, 128), jnp.float32),
            pltpu.VMEM((lanes,), jnp.float32),
            pltpu.SemaphoreType.DMA,
        ],
    )
    def sc_gather(flat_hbm, idx_hbm, lab_hbm, keep_hbm, minv_hbm, out_hbm,
                  idx_v, lab_v, keep_v, minv_v, ibuf, vbuf, accv, sem):
        wid = lax.axis_index("s") * nc + lax.axis_index("c")
        pltpu.sync_copy(idx_hbm, idx_v)
        pltpu.sync_copy(lab_hbm, lab_v.at[pl.ds(0, b)])
        pltpu.sync_copy(keep_hbm, keep_v)
        pltpu.sync_copy(minv_hbm, minv_v.at[pl.ds(0, b)])

        def row_body(r, acc):
            i = wid * rpw + r
            li = jnp.full((lanes,), lab_v[pl.ds(i, lanes)][0], jnp.int32)
            mi = jnp.full((lanes,), minv_v[pl.ds(i, lanes)][0], jnp.float32)
            base = jnp.full((lanes,), i * v, jnp.int32)
            for c in range(nch):
                ch = idx_v[pl.ds(c * lanes, lanes)] + base
                ibuf[c * lanes // 128, pl.ds((c * lanes) % 128, lanes)] = ch
            copies = [
                pltpu.async_copy(flat_hbm.at[ibuf.at[g]], vbuf.at[g], sem)
                for g in range(nstream)
            ]
            for cp in copies:
                cp.wait()
            for c in range(nch):
                val = vbuf[c * lanes // 128, pl.ds((c * lanes) % 128, lanes)]
                lj = lab_v[pl.ds(c * lanes, lanes)]
                kf = keep_v[pl.ds(c * lanes, lanes)]
                w = jnp.where(lj == li, kf * mi, 0.0)
                acc = acc + val * w
            return acc

        acc = lax.fori_loop(0, rpw, row_body,
                            jnp.zeros((lanes,), jnp.float32))
        accv[...] = acc
        pltpu.sync_copy(accv, out_hbm.at[wid])

    return sc_gather


def _lse_body(x_ref, out_ref):
    x = x_ref[...]
    mx = jnp.max(x, axis=1, keepdims=True)
    s = jnp.sum(jnp.exp(x - mx), axis=1, keepdims=True)
    out_ref[...] = mx + jnp.log(s)


def _lse(features, bt):
    b, v = features.shape
    return pl.pallas_call(
        _lse_body,
        grid=(b // bt,),
        in_specs=[pl.BlockSpec((bt, v), lambda r: (r, 0))],
        out_specs=pl.BlockSpec((bt, 1), lambda r: (r, 0)),
        out_shape=jax.ShapeDtypeStruct((b, 1), jnp.float32),
    )(features)


def _combine_body(lz_ref, part_ref, out_ref):
    b = lz_ref.shape[0]
    total = jnp.sum(lz_ref[...]) - jnp.sum(part_ref[...])
    out_ref[...] = jnp.full((1, 1), 1.0 / b) * total


def _combine(logz, partials):
    return pl.pallas_call(
        _combine_body,
        out_shape=jax.ShapeDtypeStruct((1, 1), jnp.float32),
    )(logz, partials)


def kernel(features, indices, labels):
    b, v = features.shape
    keep, minv = _prep(labels, indices)
    flat = features.reshape(b * v)
    partials = _make_sc_gather(b, v)(flat, indices, labels, keep, minv)
    logz = _lse(features, bt=32)
    loss = _combine(logz, partials)
    return loss.reshape(())


# speedup vs baseline: 8.8828x; 1.6978x over previous
import functools

import jax
import jax.numpy as jnp
from jax import lax
from jax.experimental import pallas as pl
from jax.experimental.pallas import tpu as pltpu
from jax.experimental.pallas import tpu_sc as plsc


def _prep_body(lab_row_ref, lab_col_ref, idx_row_ref, idx_col_ref,
               keep_ref, minv_ref):
    lab_row = lab_row_ref[...]
    lab_col = lab_col_ref[...]
    idx_row = idx_row_ref[...]
    idx_col = idx_col_ref[...]
    b = lab_row.shape[1]
    eq_lab = lab_col == lab_row
    eq_idx = idx_col == idx_row
    ia = lax.broadcasted_iota(jnp.int32, (b, b), 0)
    ij = lax.broadcasted_iota(jnp.int32, (b, b), 1)
    dup = eq_lab & eq_idx & (ia < ij)
    keep = jnp.where(jnp.any(dup, axis=0, keepdims=True), 0.0, 1.0)
    m = jnp.sum(eq_lab.astype(jnp.float32) * keep, axis=1, keepdims=True)
    keep_ref[...] = keep
    minv_ref[...] = 1.0 / m


def _prep(labels, indices):
    b = labels.shape[0]
    keep, minv = pl.pallas_call(
        _prep_body,
        out_shape=[jax.ShapeDtypeStruct((1, b), jnp.float32),
                   jax.ShapeDtypeStruct((b, 1), jnp.float32)],
    )(labels.reshape(1, b), labels.reshape(b, 1),
      indices.reshape(1, b), indices.reshape(b, 1))
    return keep.reshape(b), minv


def _make_sc_route(b, bt):
    info = plsc.get_sparse_core_info()
    nc, ns, lanes = info.num_cores, info.num_subcores, info.num_lanes
    nw = nc * ns
    nch = b // lanes
    mesh = plsc.VectorSubcoreMesh(core_axis_name="c", subcore_axis_name="s")

    @functools.partial(
        pl.kernel, mesh=mesh,
        out_type=[jax.ShapeDtypeStruct((nw, lanes), jnp.int32),
                  jax.ShapeDtypeStruct((nw, b), jnp.int32),
                  jax.ShapeDtypeStruct((nw, b), jnp.int32)],
        scratch_types=[
            pltpu.VMEM((b,), jnp.int32),
            pltpu.VMEM((b + 16,), jnp.int32),
            pltpu.VMEM((b,), jnp.float32),
            pltpu.VMEM((b + 16,), jnp.int32),
            pltpu.VMEM((b + 16,), jnp.int32),
            pltpu.VMEM((lanes,), jnp.int32),
        ],
    )
    def sc_route(idx_hbm, lab_hbm, keep_hbm, cnt_hbm, cj_hbm, lj_hbm,
                 idx_v, lab_v, keep_v, cj_v, lj_v, cnt_v):
        wid = lax.axis_index("s") * nc + lax.axis_index("c")
        pltpu.sync_copy(idx_hbm, idx_v)
        pltpu.sync_copy(lab_hbm, lab_v.at[pl.ds(0, b)])
        pltpu.sync_copy(keep_hbm, keep_v)

        base = wid * bt
        bl = [jnp.full((lanes,), lab_v[pl.ds(base + t, lanes)][0], jnp.int32)
              for t in range(bt)]

        def chunk_body(c, cur):
            lc = lab_v[pl.ds(c * lanes, lanes)]
            ic = idx_v[pl.ds(c * lanes, lanes)]
            kc = keep_v[pl.ds(c * lanes, lanes)]
            mem = jnp.where(lc == bl[0], 1, 0)
            for t in range(1, bt):
                mem = jnp.maximum(mem, jnp.where(lc == bl[t], 1, 0))
            mski = jnp.where(kc > 0.0, mem, 0)
            for t in range(lanes):
                cj_v[pl.ds(cur, lanes)] = jnp.full((lanes,), ic[t], jnp.int32)
                lj_v[pl.ds(cur, lanes)] = jnp.full((lanes,), lc[t], jnp.int32)
                cur = cur + mski[t]
            return cur

        total = lax.fori_loop(0, nch, chunk_body, jnp.int32(0))
        cnt_v[...] = jnp.full((lanes,), total, jnp.int32)
        pltpu.sync_copy(cnt_v, cnt_hbm.at[wid])
        pltpu.sync_copy(cj_v.at[pl.ds(0, b)], cj_hbm.at[wid])
        pltpu.sync_copy(lj_v.at[pl.ds(0, b)], lj_hbm.at[wid])

    return sc_route


def _main_body(cnt_ref, cj_ref, lj_ref, lab_ref, minv_ref, x_ref,
               logz_ref, s_ref):
    bt = x_ref.shape[0]
    x = x_ref[...]
    mx = jnp.max(x, axis=1, keepdims=True)
    ssum = jnp.sum(jnp.exp(x - mx), axis=1, keepdims=True)
    logz_ref[...] = mx + jnp.log(ssum)

    lab_blk = lab_ref[...]
    minv_blk = minv_ref[...]
    n = cnt_ref[0, 0, 0]
    lane_iota = lax.broadcasted_iota(jnp.int32, (bt, 128), 1)

    def body_k(k, acc):
        c = cj_ref[0, 0, k]
        lab_j = lj_ref[0, 0, k]
        start = pl.multiple_of((c // 128) * 128, 128)
        xt = x_ref[:, pl.ds(start, 128)]
        wcol = jnp.where(lab_blk == lab_j, minv_blk, 0.0)
        return acc + jnp.where(lane_iota == c % 128, xt * wcol, 0.0)

    acc = lax.fori_loop(0, n, body_k, jnp.zeros((bt, 128), jnp.float32))
    s_ref[...] = jnp.sum(acc, axis=1, keepdims=True)


def _main(features, counts, cj, lj, labels, minv, bt):
    b, v = features.shape
    nblk = b // bt
    return pl.pallas_call(
        _main_body,
        grid=(nblk,),
        in_specs=[
            pl.BlockSpec((1, 1, 16), lambda r: (r, 0, 0),
                         memory_space=pltpu.SMEM),
            pl.BlockSpec((1, 1, b), lambda r: (r, 0, 0),
                         memory_space=pltpu.SMEM),
            pl.BlockSpec((1, 1, b), lambda r: (r, 0, 0),
                         memory_space=pltpu.SMEM),
            pl.BlockSpec((bt, 1), lambda r: (r, 0)),
            pl.BlockSpec((bt, 1), lambda r: (r, 0)),
            pl.BlockSpec((bt, v), lambda r: (r, 0)),
        ],
        out_specs=[pl.BlockSpec((bt, 1), lambda r: (r, 0)),
                   pl.BlockSpec((bt, 1), lambda r: (r, 0))],
        out_shape=[jax.ShapeDtypeStruct((b, 1), jnp.float32),
                   jax.ShapeDtypeStruct((b, 1), jnp.float32)],
    )(counts.reshape(nblk, 1, 16), cj.reshape(nblk, 1, b),
      lj.reshape(nblk, 1, b), labels.reshape(b, 1), minv, features)


def _combine_body(lz_ref, s_ref, out_ref):
    b = lz_ref.shape[0]
    total = jnp.sum(lz_ref[...]) - jnp.sum(s_ref[...])
    out_ref[...] = jnp.full((1, 1), 1.0 / b) * total


def _combine(logz, svec):
    return pl.pallas_call(
        _combine_body,
        out_shape=jax.ShapeDtypeStruct((1, 1), jnp.float32),
    )(logz, svec)


def kernel(features, indices, labels):
    b, v = features.shape
    bt = 32
    keep, minv = _prep(labels, indices)
    counts, cj, lj = _make_sc_route(b, bt)(indices, labels, keep)
    logz, svec = _main(features, counts, cj, lj, labels, minv, bt)
    loss = _combine(logz, svec)
    return loss.reshape(())
